# baseline (speedup 1.0000x reference)
import functools

import jax
import jax.numpy as jnp
from jax.experimental import pallas as pl

D = 128
ALPHA = 0.2
BM = 400


def _support_body(x_ref, w_ref, o_ref):
    o_ref[...] = jnp.dot(
        x_ref[...], w_ref[...], preferred_element_type=jnp.float32
    ).astype(jnp.bfloat16)


def _support(x, W):
    return pl.pallas_call(
        _support_body,
        out_shape=jax.ShapeDtypeStruct((x.shape[0], D), jnp.bfloat16),
    )(x, W)


def _layer_body(adj_ref, s_ref, b_ref, o_ref, *, slope):
    a = adj_ref[...].astype(jnp.bfloat16)
    acc = jnp.dot(a, s_ref[...], preferred_element_type=jnp.float32)
    acc = acc + b_ref[...]
    o_ref[...] = jnp.where(acc > 0, acc, acc * slope)


def _gcn_layer(adj, support, b, slope):
    n_out, k = adj.shape
    return pl.pallas_call(
        functools.partial(_layer_body, slope=slope),
        grid=(n_out // BM,),
        in_specs=[
            pl.BlockSpec((BM, k), lambda i: (i, 0)),
            pl.BlockSpec((k, D), lambda i: (0, 0)),
            pl.BlockSpec((1, D), lambda i: (0, 0)),
        ],
        out_specs=pl.BlockSpec((BM, D), lambda i: (i, 0)),
        out_shape=jax.ShapeDtypeStruct((n_out, D), jnp.float32),
    )(adj, support, b)


def kernel(ufea, vfea, UV_adj, VU_adj, W1, W2, W3, W4, b1, b2, b3, b4):
    s1 = _support(ufea, W1)
    s2 = _support(vfea, W2)
    U1 = _gcn_layer(VU_adj, s1, b1.reshape(1, D), ALPHA)
    I1 = _gcn_layer(UV_adj, s2, b2.reshape(1, D), ALPHA)
    s3 = _support(U1, W3)
    s4 = _support(I1, W4)
    U_out = _gcn_layer(UV_adj, s3, b3.reshape(1, D), 0.0)
    I_out = _gcn_layer(VU_adj, s4, b4.reshape(1, D), 0.0)
    return U_out, I_out


# speedup vs baseline: 1.3098x; 1.3098x over previous
import functools

import jax
import jax.numpy as jnp
from jax.experimental import pallas as pl

D = 128
ALPHA = 0.2
BM = 400


def _support_body(x_ref, w_ref, o_ref):
    o_ref[...] = jnp.dot(
        x_ref[...], w_ref[...], preferred_element_type=jnp.float32
    ).astype(jnp.bfloat16)


def _support(x, W):
    return pl.pallas_call(
        _support_body,
        out_shape=jax.ShapeDtypeStruct((x.shape[0], D), jnp.bfloat16),
    )(x, W)


def _support2_body(x2_ref, w2_ref, x3_ref, w3_ref, o_ref):
    o_ref[:, :D] = jnp.dot(
        x2_ref[...], w2_ref[...], preferred_element_type=jnp.float32
    ).astype(jnp.bfloat16)
    o_ref[:, D:] = jnp.dot(
        x3_ref[...], w3_ref[...], preferred_element_type=jnp.float32
    ).astype(jnp.bfloat16)


def _support2(x2, W2, x3, W3):
    return pl.pallas_call(
        _support2_body,
        out_shape=jax.ShapeDtypeStruct((x2.shape[0], 2 * D), jnp.bfloat16),
    )(x2, W2, x3, W3)


def _layer_body(adj_ref, s_ref, b_ref, o_ref, *, slope):
    a = adj_ref[...].astype(jnp.bfloat16)
    acc = jnp.dot(a, s_ref[...], preferred_element_type=jnp.float32)
    acc = acc + b_ref[...]
    o_ref[...] = jnp.where(acc > 0, acc, acc * slope)


def _gcn_layer(adj, support, b, slope):
    n_out, k = adj.shape
    return pl.pallas_call(
        functools.partial(_layer_body, slope=slope),
        grid=(n_out // BM,),
        in_specs=[
            pl.BlockSpec((BM, k), lambda i: (i, 0)),
            pl.BlockSpec((k, D), lambda i: (0, 0)),
            pl.BlockSpec((1, D), lambda i: (0, 0)),
        ],
        out_specs=pl.BlockSpec((BM, D), lambda i: (i, 0)),
        out_shape=jax.ShapeDtypeStruct((n_out, D), jnp.float32),
    )(adj, support, b)


def _dual_body(adj_ref, s_ref, b_ref, o1_ref, o2_ref):
    a = adj_ref[...].astype(jnp.bfloat16)
    acc = jnp.dot(a, s_ref[...], preferred_element_type=jnp.float32)
    acc = acc + b_ref[...]
    a1 = acc[:, :D]
    a2 = acc[:, D:]
    o1_ref[...] = jnp.where(a1 > 0, a1, a1 * ALPHA)
    o2_ref[...] = jnp.maximum(a2, 0.0)


def _gcn_dual_layer(adj, support2, b2b3):
    n_out, k = adj.shape
    return pl.pallas_call(
        _dual_body,
        grid=(n_out // BM,),
        in_specs=[
            pl.BlockSpec((BM, k), lambda i: (i, 0)),
            pl.BlockSpec((k, 2 * D), lambda i: (0, 0)),
            pl.BlockSpec((1, 2 * D), lambda i: (0, 0)),
        ],
        out_specs=[
            pl.BlockSpec((BM, D), lambda i: (i, 0)),
            pl.BlockSpec((BM, D), lambda i: (i, 0)),
        ],
        out_shape=[
            jax.ShapeDtypeStruct((n_out, D), jnp.float32),
            jax.ShapeDtypeStruct((n_out, D), jnp.float32),
        ],
    )(adj, support2, b2b3)


def kernel(ufea, vfea, UV_adj, VU_adj, W1, W2, W3, W4, b1, b2, b3, b4):
    s1 = _support(ufea, W1)
    U1 = _gcn_layer(VU_adj, s1, b1.reshape(1, D), ALPHA)
    s23 = _support2(vfea, W2, U1, W3)
    b23 = jnp.concatenate([b2, b3]).reshape(1, 2 * D)
    I1, U_out = _gcn_dual_layer(UV_adj, s23, b23)
    s4 = _support(I1, W4)
    I_out = _gcn_layer(VU_adj, s4, b4.reshape(1, D), 0.0)
    return U_out, I_out
